# baseline (speedup 1.0000x reference)
import functools

import jax
import jax.numpy as jnp
from jax import lax
from jax.experimental import pallas as pl
from jax.experimental.pallas import tpu as pltpu

BATCH = 1024
SEQ = 50
VOCAB = 1000
BB = 64
GRID = BATCH // BB


def _onehot_tc_body(ann_ref, out_ref):
    ann = ann_ref[...]
    cols = lax.broadcasted_iota(jnp.int32, (BB, SEQ, VOCAB), 2)
    hit = (cols == ann[:, :, None]) & (ann[:, :, None] != 0)
    out_ref[...] = hit.astype(jnp.float32)


@jax.jit
def _onehot_tc(ann):
    return pl.pallas_call(
        _onehot_tc_body,
        grid=(GRID,),
        in_specs=[pl.BlockSpec((BB, SEQ), lambda i: (i, 0))],
        out_specs=pl.BlockSpec((BB, SEQ, VOCAB), lambda i: (i, 0, 0)),
        out_shape=jax.ShapeDtypeStruct((BATCH, SEQ, VOCAB), jnp.float32),
        compiler_params=pltpu.CompilerParams(
            dimension_semantics=("arbitrary",),
        ),
    )(ann)


def kernel(annotation, alignment, W):
    del alignment, W
    return _onehot_tc(annotation.astype(jnp.int32))


# speedup vs baseline: 1.0009x; 1.0009x over previous
import functools

import jax
import jax.numpy as jnp
from jax import lax
from jax.experimental import pallas as pl
from jax.experimental.pallas import tpu as pltpu

BATCH = 1024
SEQ = 50
VOCAB = 1000
BB = 32
NCHUNK = BATCH // BB
NBUF = 4


def _onehot_tc_body(ann_ref, out_ref, *scratch):
    bufs = scratch[:NBUF]
    sems = scratch[NBUF:]
    cols = lax.broadcasted_iota(jnp.int32, (BB, SEQ, VOCAB), 2)

    def chunk(c, buf):
        a = ann_ref[pl.ds(c * BB, BB), :][:, :, None]
        buf[...] = ((cols == a) & (a != 0)).astype(jnp.float32)

    def fire(c, b):
        pltpu.make_async_copy(
            bufs[b], out_ref.at[pl.ds(c * BB, BB)], sems[b]
        ).start()

    def wait(b):
        pltpu.make_async_copy(
            bufs[b], out_ref.at[pl.ds(0, BB)], sems[b]
        ).wait()

    for b in range(NBUF):
        chunk(b, bufs[b])
        fire(b, b)

    def step(t, _):
        for b in range(NBUF):
            c = t * NBUF + b
            wait(b)
            chunk(c, bufs[b])
            fire(c, b)
        return 0
    lax.fori_loop(1, NCHUNK // NBUF, step, 0)

    for b in range(NBUF):
        wait(b)


@jax.jit
def _onehot_tc(ann):
    return pl.pallas_call(
        _onehot_tc_body,
        in_specs=[pl.BlockSpec(memory_space=pltpu.MemorySpace.VMEM)],
        out_specs=pl.BlockSpec(memory_space=pltpu.MemorySpace.HBM),
        out_shape=jax.ShapeDtypeStruct((BATCH, SEQ, VOCAB), jnp.float32),
        scratch_shapes=(
            [pltpu.VMEM((BB, SEQ, VOCAB), jnp.float32) for _ in range(NBUF)]
            + [pltpu.SemaphoreType.DMA for _ in range(NBUF)]
        ),
    )(ann)


def kernel(annotation, alignment, W):
    del alignment, W
    return _onehot_tc(annotation.astype(jnp.int32))
